# Initial kernel scaffold; baseline (speedup 1.0000x reference)
import jax
import jax.numpy as jnp
from jax.experimental import pallas as pl


def kernel(h, frac_coords, lattices, frac_diff, W_e1, b_e1, W_e2, b_e2, W_n1, b_n1, W_n2, b_n2, ln_g, ln_b, edge_index, edge2graph):
    raise NotImplementedError("write your pallas kernel here")


# speedup vs baseline: 2.1349x; 2.1349x over previous
import functools

import jax
import jax.numpy as jnp
import numpy as np
from jax import lax
from jax.experimental import pallas as pl
from jax.experimental.pallas import tpu as pltpu
from jax.experimental.pallas import tpu_sc as plsc

N = 10000
E = 320000
G = 64
H = 128
NF = 32

_HIGH = jax.lax.Precision.HIGHEST


def _dot(a, b):
    return jnp.dot(a, b, precision=_HIGH, preferred_element_type=jnp.float32)


def _silu(x):
    return x / (1.0 + jnp.exp(-x))


_NB1 = 2000


def _s1_body(h_ref, g_ref, b_ref, wa_ref, wb_ref, hn_ref, p1_ref, p2_ref):
    x = h_ref[...]
    mu = jnp.mean(x, axis=1, keepdims=True)
    var = jnp.mean((x - mu) ** 2, axis=1, keepdims=True)
    hn = (x - mu) * lax.rsqrt(var + 1e-5) * g_ref[...] + b_ref[...]
    hn_ref[...] = hn
    p1_ref[...] = _dot(hn, wa_ref[...])
    p2_ref[...] = _dot(hn, wb_ref[...])


def _stage1(h, ln_g, ln_b, wa, wb):
    nblk = N // _NB1
    return pl.pallas_call(
        _s1_body,
        grid=(nblk,),
        in_specs=[
            pl.BlockSpec((_NB1, H), lambda i: (i, 0)),
            pl.BlockSpec((1, H), lambda i: (0, 0)),
            pl.BlockSpec((1, H), lambda i: (0, 0)),
            pl.BlockSpec((H, H), lambda i: (0, 0)),
            pl.BlockSpec((H, H), lambda i: (0, 0)),
        ],
        out_specs=[
            pl.BlockSpec((_NB1, H), lambda i: (i, 0)),
            pl.BlockSpec((_NB1, H), lambda i: (i, 0)),
            pl.BlockSpec((_NB1, H), lambda i: (i, 0)),
        ],
        out_shape=[jax.ShapeDtypeStruct((N, H), jnp.float32)] * 3,
    )(h, ln_g, ln_b, wa, wb)


def _stage2(p1, p2, src, dst):
    info = plsc.get_sparse_core_info()
    nc, ns = info.num_cores, info.num_subcores
    nw = nc * ns
    ew = E // nw
    c = 80
    nch = ew // c
    mesh = plsc.VectorSubcoreMesh(core_axis_name="c", subcore_axis_name="s")

    @functools.partial(
        pl.kernel,
        out_type=[jax.ShapeDtypeStruct((E, H), jnp.float32)] * 2,
        mesh=mesh,
        scratch_types=[
            pltpu.VMEM((c,), jnp.int32),
            pltpu.VMEM((c,), jnp.int32),
            pltpu.VMEM((c, H), jnp.float32),
            pltpu.VMEM((c, H), jnp.float32),
            pltpu.SemaphoreType.DMA,
            pltpu.SemaphoreType.DMA,
        ],
    )
    def k(p1_hbm, p2_hbm, src_hbm, dst_hbm, pa_hbm, pb_hbm,
          srcv, dstv, bufa, bufb, sema, semb):
        wid = lax.axis_index("s") * nc + lax.axis_index("c")
        base = wid * ew

        def chunk(i, carry):
            off = base + i * c
            pltpu.sync_copy(src_hbm.at[pl.ds(off, c)], srcv)
            pltpu.sync_copy(dst_hbm.at[pl.ds(off, c)], dstv)
            cpa = pltpu.async_copy(p1_hbm.at[srcv], bufa, sema)
            cpb = pltpu.async_copy(p2_hbm.at[dstv], bufb, semb)
            cpa.wait()
            cpb.wait()
            pltpu.sync_copy(bufa, pa_hbm.at[pl.ds(off, c)])
            pltpu.sync_copy(bufb, pb_hbm.at[pl.ds(off, c)])
            return carry

        lax.fori_loop(0, nch, chunk, 0)

    return k(p1, p2, src, dst)


_BE = 3200


def _s3_body(pa_ref, pb_ref, fd_ref, eg_ref, lat_ref, wc_ref, be1_ref,
             wd_ref, w2_ref, b2_ref, out_ref):
    q = jnp.broadcast_to(be1_ref[...], (G, H))
    for i in range(3):
        for j in range(3):
            ip = (lat_ref[:, 3 * i + 0:3 * i + 1] * lat_ref[:, 3 * j + 0:3 * j + 1]
                  + lat_ref[:, 3 * i + 1:3 * i + 2] * lat_ref[:, 3 * j + 1:3 * j + 2]
                  + lat_ref[:, 3 * i + 2:3 * i + 3] * lat_ref[:, 3 * j + 2:3 * j + 3])
            q = q + ip * wc_ref[3 * i + j:3 * i + j + 1, :]
    eg = eg_ref[...]
    onehot = (eg == lax.broadcasted_iota(jnp.int32, (1, G), 1)).astype(jnp.float32)
    zq = _dot(onehot, q)
    fd = fd_ref[...]
    freqs = 2.0 * np.pi * lax.broadcasted_iota(jnp.int32, (1, NF), 1).astype(jnp.float32)
    theta = jnp.concatenate([fd[:, d:d + 1] * freqs for d in range(3)], axis=1)
    emb = jnp.concatenate([jnp.sin(theta), jnp.cos(theta)], axis=1)
    z = pa_ref[...] + pb_ref[...] + zq + _dot(emb, wd_ref[...])
    t = _silu(z)
    out_ref[...] = _silu(_dot(t, w2_ref[...]) + b2_ref[...])


def _stage3(pa, pb, fd, eg2, lat9, wc, be1, wd, w2, b2):
    nblk = E // _BE
    return pl.pallas_call(
        _s3_body,
        grid=(nblk,),
        in_specs=[
            pl.BlockSpec((_BE, H), lambda i: (i, 0)),
            pl.BlockSpec((_BE, H), lambda i: (i, 0)),
            pl.BlockSpec((_BE, 3), lambda i: (i, 0)),
            pl.BlockSpec((_BE, 1), lambda i: (i, 0)),
            pl.BlockSpec((G, 9), lambda i: (0, 0)),
            pl.BlockSpec((9, H), lambda i: (0, 0)),
            pl.BlockSpec((1, H), lambda i: (0, 0)),
            pl.BlockSpec((2 * 3 * NF, H), lambda i: (0, 0)),
            pl.BlockSpec((H, H), lambda i: (0, 0)),
            pl.BlockSpec((1, H), lambda i: (0, 0)),
        ],
        out_specs=pl.BlockSpec((_BE, H), lambda i: (i, 0)),
        out_shape=jax.ShapeDtypeStruct((E, H), jnp.float32),
    )(pa, pb, fd, eg2, lat9, wc, be1, wd, w2, b2)


def _stage4(e, src, zeros_h):
    info = plsc.get_sparse_core_info()
    nc, ns = info.num_cores, info.num_subcores
    nw = nc * ns
    ew = E // nw
    c = 80
    nch = ew // c
    mesh = plsc.VectorSubcoreMesh(core_axis_name="c", subcore_axis_name="s")

    @functools.partial(
        pl.kernel,
        out_type=[
            jax.ShapeDtypeStruct((N, H), jnp.float32),
            jax.ShapeDtypeStruct((N, H), jnp.float32),
        ],
        mesh=mesh,
        scratch_types=[
            pltpu.VMEM_SHARED((N, H), jnp.float32),
            pltpu.VMEM((c,), jnp.int32),
            pltpu.VMEM((c, H), jnp.float32),
        ],
    )
    def k(e_hbm, src_hbm, zh_hbm, s0_hbm, s1_hbm, acc_s, idxv, ebuf):
        cid = lax.axis_index("c")
        sid = lax.axis_index("s")
        wid = sid * nc + cid
        base = wid * ew

        nzc = N // 128
        for kk in range(nzc // ns + 1):
            j = sid + kk * ns

            @pl.when(j < nzc)
            def _():
                pltpu.sync_copy(zh_hbm.at[pl.ds(j * 128, 128)],
                                acc_s.at[pl.ds(j * 128, 128)])

        @pl.when(sid == 0)
        def _():
            pltpu.sync_copy(zh_hbm.at[pl.ds(nzc * 128, N - nzc * 128)],
                            acc_s.at[pl.ds(nzc * 128, N - nzc * 128)])

        plsc.subcore_barrier()

        def chunk(i, carry):
            off = base + i * c
            pltpu.sync_copy(src_hbm.at[pl.ds(off, c)], idxv)
            pltpu.sync_copy(e_hbm.at[pl.ds(off, c)], ebuf)
            pltpu.sync_copy(ebuf, acc_s.at[idxv], add=True)
            return carry

        lax.fori_loop(0, nch, chunk, 0)
        plsc.subcore_barrier()

        def dump(r0, nr):
            @pl.when(cid == 0)
            def _():
                pltpu.sync_copy(acc_s.at[pl.ds(r0, nr)],
                                s0_hbm.at[pl.ds(r0, nr)])

            @pl.when(cid == 1)
            def _():
                pltpu.sync_copy(acc_s.at[pl.ds(r0, nr)],
                                s1_hbm.at[pl.ds(r0, nr)])

        for kk in range(nzc // ns + 1):
            j = sid + kk * ns

            @pl.when(j < nzc)
            def _():
                dump(j * 128, 128)

        @pl.when(sid == 0)
        def _():
            dump(nzc * 128, N - nzc * 128)

    return k(e, src, zeros_h)


def _stage4b(src, zeros_h, ones_h):
    info = plsc.get_sparse_core_info()
    nc, ns = info.num_cores, info.num_subcores
    nw = nc * ns
    ew = E // nw
    c = 80
    nch = ew // c
    mesh = plsc.VectorSubcoreMesh(core_axis_name="c", subcore_axis_name="s")

    @functools.partial(
        pl.kernel,
        out_type=[
            jax.ShapeDtypeStruct((N, H), jnp.float32),
            jax.ShapeDtypeStruct((N, H), jnp.float32),
        ],
        mesh=mesh,
        scratch_types=[
            pltpu.VMEM_SHARED((N, H), jnp.float32),
            pltpu.VMEM((c,), jnp.int32),
            pltpu.VMEM((c, H), jnp.float32),
        ],
    )
    def k(src_hbm, zh_hbm, ones_hbm, c0_hbm, c1_hbm, acc_s, idxv, obuf):
        cid = lax.axis_index("c")
        sid = lax.axis_index("s")
        wid = sid * nc + cid
        base = wid * ew
        pltpu.sync_copy(ones_hbm, obuf)

        nzc = N // 128
        for kk in range(nzc // ns + 1):
            j = sid + kk * ns

            @pl.when(j < nzc)
            def _():
                pltpu.sync_copy(zh_hbm.at[pl.ds(j * 128, 128)],
                                acc_s.at[pl.ds(j * 128, 128)])

        @pl.when(sid == 0)
        def _():
            pltpu.sync_copy(zh_hbm.at[pl.ds(nzc * 128, N - nzc * 128)],
                            acc_s.at[pl.ds(nzc * 128, N - nzc * 128)])

        plsc.subcore_barrier()

        def chunk(i, carry):
            off = base + i * c
            pltpu.sync_copy(src_hbm.at[pl.ds(off, c)], idxv)
            pltpu.sync_copy(obuf, acc_s.at[idxv], add=True)
            return carry

        lax.fori_loop(0, nch, chunk, 0)
        plsc.subcore_barrier()

        def dump(r0, nr):
            @pl.when(cid == 0)
            def _():
                pltpu.sync_copy(acc_s.at[pl.ds(r0, nr)],
                                c0_hbm.at[pl.ds(r0, nr)])

            @pl.when(cid == 1)
            def _():
                pltpu.sync_copy(acc_s.at[pl.ds(r0, nr)],
                                c1_hbm.at[pl.ds(r0, nr)])

        for kk in range(nzc // ns + 1):
            j = sid + kk * ns

            @pl.when(j < nzc)
            def _():
                dump(j * 128, 128)

        @pl.when(sid == 0)
        def _():
            dump(nzc * 128, N - nzc * 128)

    return k(src, zeros_h, ones_h)


_NB5 = 2000


def _s5_body(h_ref, hn_ref, s0_ref, s1_ref, c0_ref, c1_ref, w1a_ref, w1b_ref,
             b1_ref, w2_ref, b2_ref, out_ref):
    sums = s0_ref[...] + s1_ref[...]
    cnt = c0_ref[:, 0:1] + c1_ref[:, 0:1]
    agg = sums / jnp.maximum(cnt, 1.0)
    t = _silu(_dot(hn_ref[...], w1a_ref[...]) + _dot(agg, w1b_ref[...])
              + b1_ref[...])
    out_ref[...] = h_ref[...] + _silu(_dot(t, w2_ref[...]) + b2_ref[...])


def _stage5(h, hn, s0, s1, c0, c1, w1a, w1b, b1, w2, b2):
    nblk = N // _NB5
    return pl.pallas_call(
        _s5_body,
        grid=(nblk,),
        in_specs=[
            pl.BlockSpec((_NB5, H), lambda i: (i, 0)),
            pl.BlockSpec((_NB5, H), lambda i: (i, 0)),
            pl.BlockSpec((_NB5, H), lambda i: (i, 0)),
            pl.BlockSpec((_NB5, H), lambda i: (i, 0)),
            pl.BlockSpec((_NB5, H), lambda i: (i, 0)),
            pl.BlockSpec((_NB5, H), lambda i: (i, 0)),
            pl.BlockSpec((H, H), lambda i: (0, 0)),
            pl.BlockSpec((H, H), lambda i: (0, 0)),
            pl.BlockSpec((1, H), lambda i: (0, 0)),
            pl.BlockSpec((H, H), lambda i: (0, 0)),
            pl.BlockSpec((1, H), lambda i: (0, 0)),
        ],
        out_specs=pl.BlockSpec((_NB5, H), lambda i: (i, 0)),
        out_shape=jax.ShapeDtypeStruct((N, H), jnp.float32),
    )(h, hn, s0, s1, c0, c1, w1a, w1b, b1, w2, b2)


def kernel(h, frac_coords, lattices, frac_diff,
           W_e1, b_e1, W_e2, b_e2, W_n1, b_n1, W_n2, b_n2,
           ln_g, ln_b, edge_index, edge2graph):
    src = edge_index[0]
    dst = edge_index[1]
    wa = W_e1[:H]
    wb = W_e1[H:2 * H]
    wc = W_e1[2 * H:2 * H + 9]
    wd = W_e1[2 * H + 9:]
    lat9 = lattices.reshape(G, 9)
    eg2 = edge2graph.reshape(E, 1)

    hn, p1, p2 = _stage1(h, ln_g.reshape(1, H), ln_b.reshape(1, H), wa, wb)
    pa, pb = _stage2(p1, p2, src, dst)
    e = _stage3(pa, pb, frac_diff, eg2, lat9, wc, b_e1.reshape(1, H),
                wd, W_e2, b_e2.reshape(1, H))
    zeros_h = jnp.zeros((N, H), jnp.float32)
    s0, s1 = _stage4(e, src, zeros_h)
    c0, c1 = _stage4b(src, zeros_h, jnp.ones((80, H), jnp.float32))
    out = _stage5(h, hn, s0, s1, c0, c1, W_n1[:H], W_n1[H:],
                  b_n1.reshape(1, H), W_n2, b_n2.reshape(1, H))
    return out


# speedup vs baseline: 2.8577x; 1.3386x over previous
import functools

import jax
import jax.numpy as jnp
import numpy as np
from jax import lax
from jax.experimental import pallas as pl
from jax.experimental.pallas import tpu as pltpu
from jax.experimental.pallas import tpu_sc as plsc

N = 10000
E = 320000
G = 64
H = 128
NF = 32

_HIGH = jax.lax.Precision.HIGHEST


def _dot(a, b):
    return jnp.dot(a, b, precision=_HIGH, preferred_element_type=jnp.float32)


def _dotf(a, b):
    return jnp.dot(a, b, preferred_element_type=jnp.float32)


def _silu(x):
    return x / (1.0 + jnp.exp(-x))


_NB1 = 2000


def _s1_body(h_ref, g_ref, b_ref, wa_ref, wb_ref, hn_ref, p1_ref, p2_ref):
    x = h_ref[...]
    mu = jnp.mean(x, axis=1, keepdims=True)
    var = jnp.mean((x - mu) ** 2, axis=1, keepdims=True)
    hn = (x - mu) * lax.rsqrt(var + 1e-5) * g_ref[...] + b_ref[...]
    hn_ref[...] = hn
    p1_ref[...] = _dot(hn, wa_ref[...])
    p2_ref[...] = _dot(hn, wb_ref[...])


def _stage1(h, ln_g, ln_b, wa, wb):
    nblk = N // _NB1
    return pl.pallas_call(
        _s1_body,
        grid=(nblk,),
        in_specs=[
            pl.BlockSpec((_NB1, H), lambda i: (i, 0)),
            pl.BlockSpec((1, H), lambda i: (0, 0)),
            pl.BlockSpec((1, H), lambda i: (0, 0)),
            pl.BlockSpec((H, H), lambda i: (0, 0)),
            pl.BlockSpec((H, H), lambda i: (0, 0)),
        ],
        out_specs=[
            pl.BlockSpec((_NB1, H), lambda i: (i, 0)),
            pl.BlockSpec((_NB1, H), lambda i: (i, 0)),
            pl.BlockSpec((_NB1, H), lambda i: (i, 0)),
        ],
        out_shape=[jax.ShapeDtypeStruct((N, H), jnp.float32)] * 3,
    )(h, ln_g, ln_b, wa, wb)


def _stage2(p1, p2, src, dst):
    info = plsc.get_sparse_core_info()
    nc, ns = info.num_cores, info.num_subcores
    nw = nc * ns
    ew = E // nw
    c = 80
    nch = ew // c
    mesh = plsc.VectorSubcoreMesh(core_axis_name="c", subcore_axis_name="s")

    gb = 5
    gr = gb * c
    ngr = ew // gr

    @functools.partial(
        pl.kernel,
        out_type=[jax.ShapeDtypeStruct((E, H), jnp.float32)] * 2,
        mesh=mesh,
        scratch_types=[
            pltpu.VMEM((ew,), jnp.int32),
            pltpu.VMEM((ew,), jnp.int32),
            pltpu.VMEM((gr, H), jnp.float32),
            pltpu.VMEM((gr, H), jnp.float32),
            pltpu.SemaphoreType.DMA,
            pltpu.SemaphoreType.DMA,
        ],
    )
    def k(p1_hbm, p2_hbm, src_hbm, dst_hbm, pa_hbm, pb_hbm,
          srcv, dstv, bufa, bufb, sema, semb):
        wid = lax.axis_index("s") * nc + lax.axis_index("c")
        base = wid * ew
        pltpu.sync_copy(src_hbm.at[pl.ds(base, ew)], srcv)
        pltpu.sync_copy(dst_hbm.at[pl.ds(base, ew)], dstv)

        def group(g, carry):
            goff = g * gr
            cps = []
            for b in range(gb):
                o = b * c
                cps.append(pltpu.async_copy(
                    p1_hbm.at[srcv.at[pl.ds(goff + o, c)]],
                    bufa.at[pl.ds(o, c)], sema))
                cps.append(pltpu.async_copy(
                    p2_hbm.at[dstv.at[pl.ds(goff + o, c)]],
                    bufb.at[pl.ds(o, c)], semb))
            for cp in cps:
                cp.wait()
            pltpu.sync_copy(bufa, pa_hbm.at[pl.ds(base + goff, gr)])
            pltpu.sync_copy(bufb, pb_hbm.at[pl.ds(base + goff, gr)])
            return carry

        lax.fori_loop(0, ngr, group, 0)

    return k(p1, p2, src, dst)


_BE = 3200


def _s3_body(pa_ref, pb_ref, fd_ref, eg_ref, lat_ref, wc_ref, be1_ref,
             wd_ref, w2_ref, b2_ref, out_ref):
    q = jnp.broadcast_to(be1_ref[...], (G, H))
    for i in range(3):
        for j in range(3):
            ip = (lat_ref[:, 3 * i + 0:3 * i + 1] * lat_ref[:, 3 * j + 0:3 * j + 1]
                  + lat_ref[:, 3 * i + 1:3 * i + 2] * lat_ref[:, 3 * j + 1:3 * j + 2]
                  + lat_ref[:, 3 * i + 2:3 * i + 3] * lat_ref[:, 3 * j + 2:3 * j + 3])
            q = q + ip * wc_ref[3 * i + j:3 * i + j + 1, :]
    eg = eg_ref[...]
    onehot = (eg == lax.broadcasted_iota(jnp.int32, (1, G), 1)).astype(jnp.float32)
    zq = _dotf(onehot, q)
    fd = fd_ref[...]
    freqs = 2.0 * np.pi * lax.broadcasted_iota(jnp.int32, (1, NF), 1).astype(jnp.float32)
    theta = jnp.concatenate([fd[:, d:d + 1] * freqs for d in range(3)], axis=1)
    emb = jnp.concatenate([jnp.sin(theta), jnp.cos(theta)], axis=1)
    z = pa_ref[...] + pb_ref[...] + zq + _dotf(emb, wd_ref[...])
    t = _silu(z)
    out_ref[...] = _silu(_dotf(t, w2_ref[...]) + b2_ref[...])


def _stage3(pa, pb, fd, eg2, lat9, wc, be1, wd, w2, b2):
    nblk = E // _BE
    return pl.pallas_call(
        _s3_body,
        grid=(nblk,),
        in_specs=[
            pl.BlockSpec((_BE, H), lambda i: (i, 0)),
            pl.BlockSpec((_BE, H), lambda i: (i, 0)),
            pl.BlockSpec((_BE, 3), lambda i: (i, 0)),
            pl.BlockSpec((_BE, 1), lambda i: (i, 0)),
            pl.BlockSpec((G, 9), lambda i: (0, 0)),
            pl.BlockSpec((9, H), lambda i: (0, 0)),
            pl.BlockSpec((1, H), lambda i: (0, 0)),
            pl.BlockSpec((2 * 3 * NF, H), lambda i: (0, 0)),
            pl.BlockSpec((H, H), lambda i: (0, 0)),
            pl.BlockSpec((1, H), lambda i: (0, 0)),
        ],
        out_specs=pl.BlockSpec((_BE, H), lambda i: (i, 0)),
        out_shape=jax.ShapeDtypeStruct((E, H), jnp.float32),
    )(pa, pb, fd, eg2, lat9, wc, be1, wd, w2, b2)


def _stage4(e, src, zeros_h):
    info = plsc.get_sparse_core_info()
    nc, ns = info.num_cores, info.num_subcores
    nw = nc * ns
    ew = E // nw
    c = 80
    nch = ew // c
    mesh = plsc.VectorSubcoreMesh(core_axis_name="c", subcore_axis_name="s")

    @functools.partial(
        pl.kernel,
        out_type=[
            jax.ShapeDtypeStruct((N, H), jnp.float32),
            jax.ShapeDtypeStruct((N, H), jnp.float32),
        ],
        mesh=mesh,
        scratch_types=[
            pltpu.VMEM_SHARED((N, H), jnp.float32),
            pltpu.VMEM((c,), jnp.int32),
            pltpu.VMEM((c, H), jnp.float32),
        ],
    )
    def k(e_hbm, src_hbm, zh_hbm, s0_hbm, s1_hbm, acc_s, idxv, ebuf):
        cid = lax.axis_index("c")
        sid = lax.axis_index("s")
        wid = sid * nc + cid
        base = wid * ew

        nzc = N // 128
        for kk in range(nzc // ns + 1):
            j = sid + kk * ns

            @pl.when(j < nzc)
            def _():
                pltpu.sync_copy(zh_hbm.at[pl.ds(j * 128, 128)],
                                acc_s.at[pl.ds(j * 128, 128)])

        @pl.when(sid == 0)
        def _():
            pltpu.sync_copy(zh_hbm.at[pl.ds(nzc * 128, N - nzc * 128)],
                            acc_s.at[pl.ds(nzc * 128, N - nzc * 128)])

        plsc.subcore_barrier()

        def chunk(i, carry):
            off = base + i * c
            pltpu.sync_copy(src_hbm.at[pl.ds(off, c)], idxv)
            pltpu.sync_copy(e_hbm.at[pl.ds(off, c)], ebuf)
            pltpu.sync_copy(ebuf, acc_s.at[idxv], add=True)
            return carry

        lax.fori_loop(0, nch, chunk, 0)
        plsc.subcore_barrier()

        def dump(r0, nr):
            @pl.when(cid == 0)
            def _():
                pltpu.sync_copy(acc_s.at[pl.ds(r0, nr)],
                                s0_hbm.at[pl.ds(r0, nr)])

            @pl.when(cid == 1)
            def _():
                pltpu.sync_copy(acc_s.at[pl.ds(r0, nr)],
                                s1_hbm.at[pl.ds(r0, nr)])

        for kk in range(nzc // ns + 1):
            j = sid + kk * ns

            @pl.when(j < nzc)
            def _():
                dump(j * 128, 128)

        @pl.when(sid == 0)
        def _():
            dump(nzc * 128, N - nzc * 128)

    return k(e, src, zeros_h)


def _stage4b(src, zeros_h, ones_h):
    info = plsc.get_sparse_core_info()
    nc, ns = info.num_cores, info.num_subcores
    nw = nc * ns
    ew = E // nw
    c = 80
    nch = ew // c
    mesh = plsc.VectorSubcoreMesh(core_axis_name="c", subcore_axis_name="s")

    @functools.partial(
        pl.kernel,
        out_type=[
            jax.ShapeDtypeStruct((N, H), jnp.float32),
            jax.ShapeDtypeStruct((N, H), jnp.float32),
        ],
        mesh=mesh,
        scratch_types=[
            pltpu.VMEM_SHARED((N, H), jnp.float32),
            pltpu.VMEM((c,), jnp.int32),
            pltpu.VMEM((c, H), jnp.float32),
        ],
    )
    def k(src_hbm, zh_hbm, ones_hbm, c0_hbm, c1_hbm, acc_s, idxv, obuf):
        cid = lax.axis_index("c")
        sid = lax.axis_index("s")
        wid = sid * nc + cid
        base = wid * ew
        pltpu.sync_copy(ones_hbm, obuf)

        nzc = N // 128
        for kk in range(nzc // ns + 1):
            j = sid + kk * ns

            @pl.when(j < nzc)
            def _():
                pltpu.sync_copy(zh_hbm.at[pl.ds(j * 128, 128)],
                                acc_s.at[pl.ds(j * 128, 128)])

        @pl.when(sid == 0)
        def _():
            pltpu.sync_copy(zh_hbm.at[pl.ds(nzc * 128, N - nzc * 128)],
                            acc_s.at[pl.ds(nzc * 128, N - nzc * 128)])

        plsc.subcore_barrier()

        def chunk(i, carry):
            off = base + i * c
            pltpu.sync_copy(src_hbm.at[pl.ds(off, c)], idxv)
            pltpu.sync_copy(obuf, acc_s.at[idxv], add=True)
            return carry

        lax.fori_loop(0, nch, chunk, 0)
        plsc.subcore_barrier()

        def dump(r0, nr):
            @pl.when(cid == 0)
            def _():
                pltpu.sync_copy(acc_s.at[pl.ds(r0, nr)],
                                c0_hbm.at[pl.ds(r0, nr)])

            @pl.when(cid == 1)
            def _():
                pltpu.sync_copy(acc_s.at[pl.ds(r0, nr)],
                                c1_hbm.at[pl.ds(r0, nr)])

        for kk in range(nzc // ns + 1):
            j = sid + kk * ns

            @pl.when(j < nzc)
            def _():
                dump(j * 128, 128)

        @pl.when(sid == 0)
        def _():
            dump(nzc * 128, N - nzc * 128)

    return k(src, zeros_h, ones_h)


_NB5 = 2000


def _s5_body(h_ref, hn_ref, s0_ref, s1_ref, c0_ref, c1_ref, w1a_ref, w1b_ref,
             b1_ref, w2_ref, b2_ref, out_ref):
    sums = s0_ref[...] + s1_ref[...]
    cnt = c0_ref[:, 0:1] + c1_ref[:, 0:1]
    agg = sums / jnp.maximum(cnt, 1.0)
    t = _silu(_dot(hn_ref[...], w1a_ref[...]) + _dot(agg, w1b_ref[...])
              + b1_ref[...])
    out_ref[...] = h_ref[...] + _silu(_dot(t, w2_ref[...]) + b2_ref[...])


def _stage5(h, hn, s0, s1, c0, c1, w1a, w1b, b1, w2, b2):
    nblk = N // _NB5
    return pl.pallas_call(
        _s5_body,
        grid=(nblk,),
        in_specs=[
            pl.BlockSpec((_NB5, H), lambda i: (i, 0)),
            pl.BlockSpec((_NB5, H), lambda i: (i, 0)),
            pl.BlockSpec((_NB5, H), lambda i: (i, 0)),
            pl.BlockSpec((_NB5, H), lambda i: (i, 0)),
            pl.BlockSpec((_NB5, H), lambda i: (i, 0)),
            pl.BlockSpec((_NB5, H), lambda i: (i, 0)),
            pl.BlockSpec((H, H), lambda i: (0, 0)),
            pl.BlockSpec((H, H), lambda i: (0, 0)),
            pl.BlockSpec((1, H), lambda i: (0, 0)),
            pl.BlockSpec((H, H), lambda i: (0, 0)),
            pl.BlockSpec((1, H), lambda i: (0, 0)),
        ],
        out_specs=pl.BlockSpec((_NB5, H), lambda i: (i, 0)),
        out_shape=jax.ShapeDtypeStruct((N, H), jnp.float32),
    )(h, hn, s0, s1, c0, c1, w1a, w1b, b1, w2, b2)


def kernel(h, frac_coords, lattices, frac_diff,
           W_e1, b_e1, W_e2, b_e2, W_n1, b_n1, W_n2, b_n2,
           ln_g, ln_b, edge_index, edge2graph):
    src = edge_index[0]
    dst = edge_index[1]
    wa = W_e1[:H]
    wb = W_e1[H:2 * H]
    wc = W_e1[2 * H:2 * H + 9]
    wd = W_e1[2 * H + 9:]
    lat9 = lattices.reshape(G, 9)
    eg2 = edge2graph.reshape(E, 1)

    hn, p1, p2 = _stage1(h, ln_g.reshape(1, H), ln_b.reshape(1, H), wa, wb)
    pa, pb = _stage2(p1, p2, src, dst)
    e = _stage3(pa, pb, frac_diff, eg2, lat9, wc, b_e1.reshape(1, H),
                wd, W_e2, b_e2.reshape(1, H))
    zeros_h = jnp.zeros((N, H), jnp.float32)
    s0, s1 = _stage4(e, src, zeros_h)
    c0, c1 = _stage4b(src, zeros_h, jnp.ones((80, H), jnp.float32))
    out = _stage5(h, hn, s0, s1, c0, c1, W_n1[:H], W_n1[H:],
                  b_n1.reshape(1, H), W_n2, b_n2.reshape(1, H))
    return out


# speedup vs baseline: 3.0597x; 1.0707x over previous
import functools

import jax
import jax.numpy as jnp
import numpy as np
from jax import lax
from jax.experimental import pallas as pl
from jax.experimental.pallas import tpu as pltpu
from jax.experimental.pallas import tpu_sc as plsc

N = 10000
E = 320000
G = 64
H = 128
NF = 32

_HIGH = jax.lax.Precision.HIGHEST


def _dot(a, b):
    return jnp.dot(a, b, precision=_HIGH, preferred_element_type=jnp.float32)


def _dotf(a, b):
    return jnp.dot(a, b, preferred_element_type=jnp.float32)


def _silu(x):
    return x / (1.0 + jnp.exp(-x))


_NB1 = 2000


def _s1_body(h_ref, g_ref, b_ref, wa_ref, wb_ref, hn_ref, p1_ref, p2_ref):
    x = h_ref[...]
    mu = jnp.mean(x, axis=1, keepdims=True)
    var = jnp.mean((x - mu) ** 2, axis=1, keepdims=True)
    hn = (x - mu) * lax.rsqrt(var + 1e-5) * g_ref[...] + b_ref[...]
    hn_ref[...] = hn
    p1_ref[...] = _dot(hn, wa_ref[...])
    p2_ref[...] = _dot(hn, wb_ref[...])


def _stage1(h, ln_g, ln_b, wa, wb):
    nblk = N // _NB1
    return pl.pallas_call(
        _s1_body,
        grid=(nblk,),
        in_specs=[
            pl.BlockSpec((_NB1, H), lambda i: (i, 0)),
            pl.BlockSpec((1, H), lambda i: (0, 0)),
            pl.BlockSpec((1, H), lambda i: (0, 0)),
            pl.BlockSpec((H, H), lambda i: (0, 0)),
            pl.BlockSpec((H, H), lambda i: (0, 0)),
        ],
        out_specs=[
            pl.BlockSpec((_NB1, H), lambda i: (i, 0)),
            pl.BlockSpec((_NB1, H), lambda i: (i, 0)),
            pl.BlockSpec((_NB1, H), lambda i: (i, 0)),
        ],
        out_shape=[jax.ShapeDtypeStruct((N, H), jnp.float32)] * 3,
    )(h, ln_g, ln_b, wa, wb)


def _stage2(p1, p2, src, dst):
    info = plsc.get_sparse_core_info()
    nc, ns = info.num_cores, info.num_subcores
    nw = nc * ns
    ew = E // nw
    c = 80
    nch = ew // c
    mesh = plsc.VectorSubcoreMesh(core_axis_name="c", subcore_axis_name="s")

    gb = 5
    gr = gb * c
    ngr = ew // gr

    @functools.partial(
        pl.kernel,
        out_type=[jax.ShapeDtypeStruct((E, H), jnp.float32)] * 2,
        mesh=mesh,
        scratch_types=[
            pltpu.VMEM((ew,), jnp.int32),
            pltpu.VMEM((ew,), jnp.int32),
            pltpu.VMEM((gr, H), jnp.float32),
            pltpu.VMEM((gr, H), jnp.float32),
            pltpu.SemaphoreType.DMA,
            pltpu.SemaphoreType.DMA,
        ],
    )
    def k(p1_hbm, p2_hbm, src_hbm, dst_hbm, pa_hbm, pb_hbm,
          srcv, dstv, bufa, bufb, sema, semb):
        wid = lax.axis_index("s") * nc + lax.axis_index("c")
        base = wid * ew
        pltpu.sync_copy(src_hbm.at[pl.ds(base, ew)], srcv)
        pltpu.sync_copy(dst_hbm.at[pl.ds(base, ew)], dstv)

        def group(g, carry):
            goff = g * gr
            cps = []
            for b in range(gb):
                o = b * c
                cps.append(pltpu.async_copy(
                    p1_hbm.at[srcv.at[pl.ds(goff + o, c)]],
                    bufa.at[pl.ds(o, c)], sema))
                cps.append(pltpu.async_copy(
                    p2_hbm.at[dstv.at[pl.ds(goff + o, c)]],
                    bufb.at[pl.ds(o, c)], semb))
            for cp in cps:
                cp.wait()
            pltpu.sync_copy(bufa, pa_hbm.at[pl.ds(base + goff, gr)])
            pltpu.sync_copy(bufb, pb_hbm.at[pl.ds(base + goff, gr)])
            return carry

        lax.fori_loop(0, ngr, group, 0)

    return k(p1, p2, src, dst)


_BE = 3200


def _s3_body(pa_ref, pb_ref, fd_ref, eg_ref, lat_ref, wc_ref, be1_ref,
             wd_ref, w2_ref, b2_ref, out_ref):
    q = jnp.broadcast_to(be1_ref[...], (G, H))
    for i in range(3):
        for j in range(3):
            ip = (lat_ref[:, 3 * i + 0:3 * i + 1] * lat_ref[:, 3 * j + 0:3 * j + 1]
                  + lat_ref[:, 3 * i + 1:3 * i + 2] * lat_ref[:, 3 * j + 1:3 * j + 2]
                  + lat_ref[:, 3 * i + 2:3 * i + 3] * lat_ref[:, 3 * j + 2:3 * j + 3])
            q = q + ip * wc_ref[3 * i + j:3 * i + j + 1, :]
    eg = eg_ref[...]
    onehot = (eg == lax.broadcasted_iota(jnp.int32, (1, G), 1)).astype(jnp.float32)
    zq = _dotf(onehot, q)
    fd = fd_ref[...]
    freqs = 2.0 * np.pi * lax.broadcasted_iota(jnp.int32, (1, NF), 1).astype(jnp.float32)
    theta = jnp.concatenate([fd[:, d:d + 1] * freqs for d in range(3)], axis=1)
    emb = jnp.concatenate([jnp.sin(theta), jnp.cos(theta)], axis=1)
    z = pa_ref[...] + pb_ref[...] + zq + _dotf(emb, wd_ref[...])
    t = _silu(z)
    out_ref[...] = _silu(_dotf(t, w2_ref[...]) + b2_ref[...])


def _stage3(pa, pb, fd, eg2, lat9, wc, be1, wd, w2, b2):
    nblk = E // _BE
    return pl.pallas_call(
        _s3_body,
        grid=(nblk,),
        in_specs=[
            pl.BlockSpec((_BE, H), lambda i: (i, 0)),
            pl.BlockSpec((_BE, H), lambda i: (i, 0)),
            pl.BlockSpec((_BE, 3), lambda i: (i, 0)),
            pl.BlockSpec((_BE, 1), lambda i: (i, 0)),
            pl.BlockSpec((G, 9), lambda i: (0, 0)),
            pl.BlockSpec((9, H), lambda i: (0, 0)),
            pl.BlockSpec((1, H), lambda i: (0, 0)),
            pl.BlockSpec((2 * 3 * NF, H), lambda i: (0, 0)),
            pl.BlockSpec((H, H), lambda i: (0, 0)),
            pl.BlockSpec((1, H), lambda i: (0, 0)),
        ],
        out_specs=pl.BlockSpec((_BE, H), lambda i: (i, 0)),
        out_shape=jax.ShapeDtypeStruct((E, H), jnp.float32),
    )(pa, pb, fd, eg2, lat9, wc, be1, wd, w2, b2)


def _stage4(e, src, zeros_h):
    info = plsc.get_sparse_core_info()
    nc, ns = info.num_cores, info.num_subcores
    nw = nc * ns
    ew = E // nw
    c = 80
    nch = ew // c
    mesh = plsc.VectorSubcoreMesh(core_axis_name="c", subcore_axis_name="s")

    @functools.partial(
        pl.kernel,
        out_type=[
            jax.ShapeDtypeStruct((N, H), jnp.float32),
            jax.ShapeDtypeStruct((N, H), jnp.float32),
        ],
        mesh=mesh,
        scratch_types=[
            pltpu.VMEM_SHARED((N, H), jnp.float32),
            [pltpu.VMEM((c,), jnp.int32) for _ in range(4)],
            pltpu.VMEM((4 * c, H), jnp.float32),
            pltpu.SemaphoreType.DMA,
            pltpu.SemaphoreType.DMA,
        ],
    )
    def k(e_hbm, src_hbm, zh_hbm, s0_hbm, s1_hbm, acc_s, idxs, ebuf,
          seme, semi):
        cid = lax.axis_index("c")
        sid = lax.axis_index("s")
        wid = sid * nc + cid
        base = wid * ew

        nzc = N // 128
        for kk in range(nzc // ns + 1):
            j = sid + kk * ns

            @pl.when(j < nzc)
            def _():
                pltpu.sync_copy(zh_hbm.at[pl.ds(j * 128, 128)],
                                acc_s.at[pl.ds(j * 128, 128)])

        @pl.when(sid == 0)
        def _():
            pltpu.sync_copy(zh_hbm.at[pl.ds(nzc * 128, N - nzc * 128)],
                            acc_s.at[pl.ds(nzc * 128, N - nzc * 128)])

        plsc.subcore_barrier()

        def group(g, carry):
            goff = base + g * (4 * c)
            cpe = pltpu.async_copy(e_hbm.at[pl.ds(goff, 4 * c)], ebuf, seme)
            cpi = [pltpu.async_copy(src_hbm.at[pl.ds(goff + b * c, c)],
                                    idxs[b], semi) for b in range(4)]
            cpe.wait()
            for cp in cpi:
                cp.wait()
            for b in range(4):
                pltpu.sync_copy(ebuf.at[pl.ds(b * c, c)],
                                acc_s.at[idxs[b]], add=True)
            return carry

        ngr4 = nch // 4
        lax.fori_loop(0, ngr4, group, 0)
        toff = base + ngr4 * (4 * c)
        pltpu.sync_copy(src_hbm.at[pl.ds(toff, c)], idxs[0])
        pltpu.sync_copy(e_hbm.at[pl.ds(toff, c)], ebuf.at[pl.ds(0, c)])
        pltpu.sync_copy(ebuf.at[pl.ds(0, c)], acc_s.at[idxs[0]], add=True)
        plsc.subcore_barrier()

        def dump(r0, nr):
            @pl.when(cid == 0)
            def _():
                pltpu.sync_copy(acc_s.at[pl.ds(r0, nr)],
                                s0_hbm.at[pl.ds(r0, nr)])

            @pl.when(cid == 1)
            def _():
                pltpu.sync_copy(acc_s.at[pl.ds(r0, nr)],
                                s1_hbm.at[pl.ds(r0, nr)])

        for kk in range(nzc // ns + 1):
            j = sid + kk * ns

            @pl.when(j < nzc)
            def _():
                dump(j * 128, 128)

        @pl.when(sid == 0)
        def _():
            dump(nzc * 128, N - nzc * 128)

    return k(e, src, zeros_h)


def _stage4b(src, zeros_h, ones_h):
    info = plsc.get_sparse_core_info()
    nc, ns = info.num_cores, info.num_subcores
    nw = nc * ns
    ew = E // nw
    c = 80
    nch = ew // c
    mesh = plsc.VectorSubcoreMesh(core_axis_name="c", subcore_axis_name="s")

    @functools.partial(
        pl.kernel,
        out_type=[
            jax.ShapeDtypeStruct((N, H), jnp.float32),
            jax.ShapeDtypeStruct((N, H), jnp.float32),
        ],
        mesh=mesh,
        scratch_types=[
            pltpu.VMEM_SHARED((N, H), jnp.float32),
            [pltpu.VMEM((c,), jnp.int32) for _ in range(5)],
            pltpu.VMEM((c, H), jnp.float32),
            pltpu.SemaphoreType.DMA,
        ],
    )
    def k(src_hbm, zh_hbm, ones_hbm, c0_hbm, c1_hbm, acc_s, idxs, obuf,
          semi):
        cid = lax.axis_index("c")
        sid = lax.axis_index("s")
        wid = sid * nc + cid
        base = wid * ew
        pltpu.sync_copy(ones_hbm, obuf)

        nzc = N // 128
        for kk in range(nzc // ns + 1):
            j = sid + kk * ns

            @pl.when(j < nzc)
            def _():
                pltpu.sync_copy(zh_hbm.at[pl.ds(j * 128, 128)],
                                acc_s.at[pl.ds(j * 128, 128)])

        @pl.when(sid == 0)
        def _():
            pltpu.sync_copy(zh_hbm.at[pl.ds(nzc * 128, N - nzc * 128)],
                            acc_s.at[pl.ds(nzc * 128, N - nzc * 128)])

        plsc.subcore_barrier()

        def group(g, carry):
            goff = base + g * (5 * c)
            cpi = [pltpu.async_copy(src_hbm.at[pl.ds(goff + b * c, c)],
                                    idxs[b], semi) for b in range(5)]
            for cp in cpi:
                cp.wait()
            for b in range(5):
                pltpu.sync_copy(obuf, acc_s.at[idxs[b]], add=True)
            return carry

        lax.fori_loop(0, nch // 5, group, 0)
        plsc.subcore_barrier()

        def dump(r0, nr):
            @pl.when(cid == 0)
            def _():
                pltpu.sync_copy(acc_s.at[pl.ds(r0, nr)],
                                c0_hbm.at[pl.ds(r0, nr)])

            @pl.when(cid == 1)
            def _():
                pltpu.sync_copy(acc_s.at[pl.ds(r0, nr)],
                                c1_hbm.at[pl.ds(r0, nr)])

        for kk in range(nzc // ns + 1):
            j = sid + kk * ns

            @pl.when(j < nzc)
            def _():
                dump(j * 128, 128)

        @pl.when(sid == 0)
        def _():
            dump(nzc * 128, N - nzc * 128)

    return k(src, zeros_h, ones_h)


_NB5 = 2000


def _s5_body(h_ref, hn_ref, s0_ref, s1_ref, c0_ref, c1_ref, w1a_ref, w1b_ref,
             b1_ref, w2_ref, b2_ref, out_ref):
    sums = s0_ref[...] + s1_ref[...]
    cnt = c0_ref[:, 0:1] + c1_ref[:, 0:1]
    agg = sums / jnp.maximum(cnt, 1.0)
    t = _silu(_dot(hn_ref[...], w1a_ref[...]) + _dot(agg, w1b_ref[...])
              + b1_ref[...])
    out_ref[...] = h_ref[...] + _silu(_dot(t, w2_ref[...]) + b2_ref[...])


def _stage5(h, hn, s0, s1, c0, c1, w1a, w1b, b1, w2, b2):
    nblk = N // _NB5
    return pl.pallas_call(
        _s5_body,
        grid=(nblk,),
        in_specs=[
            pl.BlockSpec((_NB5, H), lambda i: (i, 0)),
            pl.BlockSpec((_NB5, H), lambda i: (i, 0)),
            pl.BlockSpec((_NB5, H), lambda i: (i, 0)),
            pl.BlockSpec((_NB5, H), lambda i: (i, 0)),
            pl.BlockSpec((_NB5, H), lambda i: (i, 0)),
            pl.BlockSpec((_NB5, H), lambda i: (i, 0)),
            pl.BlockSpec((H, H), lambda i: (0, 0)),
            pl.BlockSpec((H, H), lambda i: (0, 0)),
            pl.BlockSpec((1, H), lambda i: (0, 0)),
            pl.BlockSpec((H, H), lambda i: (0, 0)),
            pl.BlockSpec((1, H), lambda i: (0, 0)),
        ],
        out_specs=pl.BlockSpec((_NB5, H), lambda i: (i, 0)),
        out_shape=jax.ShapeDtypeStruct((N, H), jnp.float32),
    )(h, hn, s0, s1, c0, c1, w1a, w1b, b1, w2, b2)


def kernel(h, frac_coords, lattices, frac_diff,
           W_e1, b_e1, W_e2, b_e2, W_n1, b_n1, W_n2, b_n2,
           ln_g, ln_b, edge_index, edge2graph):
    src = edge_index[0]
    dst = edge_index[1]
    wa = W_e1[:H]
    wb = W_e1[H:2 * H]
    wc = W_e1[2 * H:2 * H + 9]
    wd = W_e1[2 * H + 9:]
    lat9 = lattices.reshape(G, 9)
    eg2 = edge2graph.reshape(E, 1)

    hn, p1, p2 = _stage1(h, ln_g.reshape(1, H), ln_b.reshape(1, H), wa, wb)
    pa, pb = _stage2(p1, p2, src, dst)
    e = _stage3(pa, pb, frac_diff, eg2, lat9, wc, b_e1.reshape(1, H),
                wd, W_e2, b_e2.reshape(1, H))
    zeros_h = jnp.zeros((N, H), jnp.float32)
    c0, c1 = _stage4b(src, zeros_h, jnp.ones((80, H), jnp.float32))
    s0, s1 = _stage4(e, src, zeros_h)
    out = _stage5(h, hn, s0, s1, c0, c1, W_n1[:H], W_n1[H:],
                  b_n1.reshape(1, H), W_n2, b_n2.reshape(1, H))
    return out
